# Initial kernel scaffold; baseline (speedup 1.0000x reference)
import jax
import jax.numpy as jnp
from jax.experimental import pallas as pl


def kernel(pred_s, pred_t, G, delta_x_):
    raise NotImplementedError("write your pallas kernel here")


# speedup vs baseline: 1.9015x; 1.9015x over previous
import jax
import jax.numpy as jnp
from jax.experimental import pallas as pl
from jax.experimental.pallas import tpu as pltpu

_N = 100000
_C = 128
_B = 2000
_GRID = _N // _B


def _kl_block_kernel(s_ref, t_ref, w_ref, out_ref, acc_ref):
    i = pl.program_id(0)

    @pl.when(i == 0)
    def _init():
        acc_ref[0] = 0.0
        acc_ref[1] = 0.0

    s = s_ref[...]
    t = t_ref[...]

    ms = jnp.max(s, axis=1, keepdims=True)
    ss = s - ms
    ls = ss - jnp.log(jnp.sum(jnp.exp(ss), axis=1, keepdims=True))

    mt = jnp.max(t, axis=1, keepdims=True)
    et = jnp.exp(t - mt)
    pt = et / jnp.sum(et, axis=1, keepdims=True)

    row_terms = jnp.sum(jnp.exp(pt) * (pt - ls), axis=1)
    w = w_ref[0, 0, :]

    acc_ref[0] += jnp.sum(row_terms * w)
    acc_ref[1] += jnp.sum(w)

    @pl.when(i == _GRID - 1)
    def _fini():
        out_ref[0, 0] = acc_ref[0] / jnp.maximum(acc_ref[1], 1.0)


def kernel(pred_s, pred_t, G, delta_x_):
    u = jax.random.uniform(jax.random.key(42), (_N,), dtype=jnp.float32)
    w = (u < delta_x_).astype(jnp.float32).reshape(_GRID, 1, _B)

    out = pl.pallas_call(
        _kl_block_kernel,
        grid=(_GRID,),
        in_specs=[
            pl.BlockSpec((_B, _C), lambda i: (i, 0)),
            pl.BlockSpec((_B, _C), lambda i: (i, 0)),
            pl.BlockSpec((1, 1, _B), lambda i: (i, 0, 0)),
        ],
        out_specs=pl.BlockSpec(memory_space=pltpu.SMEM),
        out_shape=jax.ShapeDtypeStruct((1, 1), jnp.float32),
        scratch_shapes=[pltpu.SMEM((2,), jnp.float32)],
    )(pred_s, pred_t, w)
    return out[0, 0]
